# baseline (speedup 1.0000x reference)
import jax
import jax.numpy as jnp
from jax import lax
from jax.experimental import pallas as pl
from jax.experimental.pallas import tpu as pltpu
from jax.experimental.pallas import tpu_sc as plsc

N = 10000
D = 128
H = 128
K = 32
NC = 2
NS = 16
NW = NC * NS
NP = 10240
C = NP // NW
SUP = 16
SUB = 4
NSUP = C // SUP
ALPHA = 0.3


def _leaky(x):
    return jnp.where(x >= 0, x, ALPHA * x)


def _q_body(x_ref, q_ref, b_ref, o_ref):
    h = jnp.dot(x_ref[...], q_ref[...], preferred_element_type=jnp.float32)
    o_ref[...] = _leaky(h + b_ref[...])


def _q_phase(x, q, qb):
    blk = 2000
    return pl.pallas_call(
        _q_body,
        grid=(N // blk,),
        in_specs=[
            pl.BlockSpec((blk, D), lambda i: (i, 0)),
            pl.BlockSpec((D, H), lambda i: (0, 0)),
            pl.BlockSpec((1, H), lambda i: (0, 0)),
        ],
        out_specs=pl.BlockSpec((blk, H), lambda i: (i, 0)),
        out_shape=jax.ShapeDtypeStruct((N, H), jnp.float32),
    )(x, q, qb.reshape(1, H))


def _bcast_lane(vec, lane):
    idx = jnp.full((16, 1), lane, jnp.int32)
    dn = lax.GatherDimensionNumbers(
        offset_dims=(), collapsed_slice_dims=(0,), start_index_map=(0,))
    return lax.gather(vec, idx, dn, (1,),
                      mode=lax.GatherScatterMode.PROMISE_IN_BOUNDS)


def _sc_body(wf_hbm, nbrf_hbm, hid_hbm, out_hbm,
             sidx0, sidx1, raw0, raw1, flat0, flat1, wv0, wv1,
             rows0, rows1, ov0, ov1, hid_sh,
             si0, si1, sw0, sw1, sr0, sr1, so0, so1, sem_h):
    sidx = (sidx0, sidx1)
    raw_v = (raw0, raw1)
    flat_v = (flat0, flat1)
    w_v = (wv0, wv1)
    rows_v = (rows0, rows1)
    out_v = (ov0, ov1)
    sem_i = (si0, si1)
    sem_w = (sw0, sw1)
    sem_r = (sr0, sr1)
    sem_o = (so0, so1)

    c = lax.axis_index("c")
    s = lax.axis_index("s")
    wid = s * NC + c
    base = wid * C

    @pl.when(s == 0)
    def _():
        pltpu.async_copy(hid_hbm, hid_sh, sem_h).wait()

    plsc.subcore_barrier()

    SK = SUP * K

    def idx_dma(m, p):
        pltpu.async_copy(
            nbrf_hbm.at[pl.ds((base + m * SUP) * K, SK)], sidx[p], sem_i[p])

    def idx_wait(p):
        pltpu.make_async_copy(
            nbrf_hbm.at[pl.ds(0, SK)], sidx[p], sem_i[p]).wait()

    def flat_compute(m, p):
        nb = base + m * SUP
        for r in range(SUP):
            roff = (nb + r) * N
            for j in range(K // 16):
                q = r * K + j * 16
                v = sidx[p][pl.ds(q, 16)]
                raw_v[p][pl.ds(q, 16)] = v
                f = jnp.minimum(v + roff, N * N - 1)
                flat_v[p][pl.ds(q, 16)] = jax.lax.shift_right_logical(f, 1)

    def fire_w(p):
        for g in range(SK // 128):
            sl = pl.ds(g * 128, 128)
            pltpu.async_copy(wf_hbm.at[flat_v[p].at[sl]],
                             w_v[p].at[sl], sem_w[p])

    def wait_w(p):
        for g in range(SK // 128):
            sl = pl.ds(g * 128, 128)
            pltpu.make_async_copy(wf_hbm.at[flat_v[p].at[sl]],
                                  w_v[p].at[sl], sem_w[p]).wait()

    def fire_rows(p, j, gp):
        sl = pl.ds(j * SUB * K, SUB * K)
        pltpu.async_copy(hid_sh.at[raw_v[p].at[sl]], rows_v[gp], sem_r[gp])

    def wait_rows(gp):
        pltpu.make_async_copy(
            hid_sh.at[raw_v[0].at[pl.ds(0, SUB * K)]],
            rows_v[gp], sem_r[gp]).wait()

    def drain_out(p):
        pltpu.make_async_copy(
            out_v[p], out_hbm.at[pl.ds(base, SUP)], sem_o[p]).wait()

    def compute_sub(m, p, j, gp):
        wait_rows(gp)

        def node(r, carry):
            nq = j * SUB + r
            q = nq * K
            rk = r * K
            roff = (base + m * SUP + nq) * N

            def getw(off):
                v32 = w_v[p][pl.ds(q + off, 16)]
                col = sidx[p][pl.ds(q + off, 16)]
                f = jnp.minimum(col + roff, N * N - 1)
                par = jax.lax.rem(f, 2)
                u = jax.lax.shift_right_logical(v32, par * 16) & 0xFFFF
                e = jax.lax.shift_right_logical(u, 7) & 0xFF
                mant = u & 0x7F
                ef = e.astype(jnp.float32)
                mf = mant.astype(jnp.float32)
                return (1.0 + mf * (1.0 / 128.0)) * jnp.exp(
                    (ef - 127.0) * 0.6931471805599453)

            w0 = getw(0)
            w1 = getw(16)
            acc = [jnp.zeros((16,), jnp.float32) for _ in range(H // 16)]
            wsum = jnp.zeros((16,), jnp.float32)
            for k in range(K):
                wk = _bcast_lane(w0 if k < 16 else w1, k % 16)
                wsum = wsum + wk
                for jj in range(H // 16):
                    acc[jj] = acc[jj] + wk * rows_v[gp][rk + k,
                                                        pl.ds(jj * 16, 16)]
            denom = wsum + 1e-6
            for jj in range(H // 16):
                out_v[p][nq, pl.ds(jj * 16, 16)] = acc[jj] / denom
            return carry

        lax.fori_loop(0, SUB, node, 0)

    def do_super(m, p):
        nb = base + m * SUP
        idx_wait(p)
        flat_compute(m, p)

        @pl.when(m >= 2)
        def _():
            drain_out(p)

        fire_rows(p, 0, 0)
        fire_w(p)
        idx_dma(m + 1, 1 - p)
        fire_rows(p, 1, 1)
        wait_w(p)
        compute_sub(m, p, 0, 0)
        fire_rows(p, 2, 0)
        compute_sub(m, p, 1, 1)
        fire_rows(p, 3, 1)
        compute_sub(m, p, 2, 0)
        compute_sub(m, p, 3, 1)
        pltpu.async_copy(out_v[p], out_hbm.at[pl.ds(nb, SUP)], sem_o[p])

    idx_dma(0, 0)

    def body(i, carry):
        do_super(2 * i, 0)
        do_super(2 * i + 1, 1)
        return carry

    lax.fori_loop(0, NSUP // 2, body, 0)
    idx_wait(0)
    drain_out(0)
    drain_out(1)


def _sc_phase(weights_flat, nbr_flat, hid):
    mesh = plsc.VectorSubcoreMesh(core_axis_name="c", subcore_axis_name="s")
    scratch = (
        [pltpu.VMEM((SUP * K,), jnp.int32) for _ in range(2)]
        + [pltpu.VMEM((SUP * K,), jnp.int32) for _ in range(4)]
        + [pltpu.VMEM((SUP * K,), jnp.int32) for _ in range(2)]
        + [pltpu.VMEM((SUB * K, H), jnp.float32) for _ in range(2)]
        + [pltpu.VMEM((SUP, H), jnp.float32) for _ in range(2)]
        + [pltpu.VMEM_SHARED((N, H), jnp.float32)]
        + [pltpu.SemaphoreType.DMA for _ in range(9)]
    )
    f = pl.kernel(
        _sc_body,
        out_type=jax.ShapeDtypeStruct((NP, H), jnp.float32),
        mesh=mesh,
        scratch_types=scratch,
    )
    return f(weights_flat, nbr_flat, hid)


def _w_body(x_ref, a_ref, w1_ref, w2_ref, b_ref, o_ref):
    h = jnp.dot(x_ref[...], w1_ref[...], preferred_element_type=jnp.float32)
    h = h + jnp.dot(a_ref[...], w2_ref[...],
                    preferred_element_type=jnp.float32)
    h = _leaky(h + b_ref[...])
    nrm = jnp.sqrt(jnp.sum(h * h, axis=1, keepdims=True)) + 1e-6
    o_ref[...] = h / nrm


def _w_phase(x, agg, w1, w2, wb):
    blk = 2000
    return pl.pallas_call(
        _w_body,
        grid=(N // blk,),
        in_specs=[
            pl.BlockSpec((blk, D), lambda i: (i, 0)),
            pl.BlockSpec((blk, H), lambda i: (i, 0)),
            pl.BlockSpec((D, H), lambda i: (0, 0)),
            pl.BlockSpec((H, H), lambda i: (0, 0)),
            pl.BlockSpec((1, H), lambda i: (0, 0)),
        ],
        out_specs=pl.BlockSpec((blk, H), lambda i: (i, 0)),
        out_shape=jax.ShapeDtypeStruct((N, H), jnp.float32),
    )(x, agg, w1, w2, wb.reshape(1, H))


def kernel(embs, weights, Q_kernel, Q_bias, W_kernel, W_bias, neighbor_set):
    x = embs[0]
    hid = _q_phase(x, Q_kernel, Q_bias)
    wf = jax.lax.bitcast_convert_type(
        weights.astype(jnp.bfloat16).reshape(N * N // 2, 2), jnp.int32)
    nbr = jnp.pad(neighbor_set.astype(jnp.int32), ((0, NP + SUP - N), (0, 0)))
    nbrf = nbr.reshape((NP + SUP) * K)
    agg = _sc_phase(wf, nbrf, hid)[:N]
    out = _w_phase(x, agg, W_kernel[:D], W_kernel[D:], W_bias)
    return out[None]


# speedup vs baseline: 50.4930x; 50.4930x over previous
import jax
import jax.numpy as jnp
from jax import lax
from jax.experimental import pallas as pl
from jax.experimental.pallas import tpu as pltpu
from jax.experimental.pallas import tpu_sc as plsc

N = 10000
D = 128
H = 128
K = 32
NC = 2
NS = 16
NW = NC * NS
NP = 10240
C = NP // NW
SUP = 16
SUB = 4
NSUP = C // SUP
ALPHA = 0.3


def _leaky(x):
    return jnp.where(x >= 0, x, ALPHA * x)


def _q_body(x_ref, q_ref, b_ref, o_ref):
    h = jnp.dot(x_ref[...], q_ref[...], preferred_element_type=jnp.float32)
    o_ref[...] = _leaky(h + b_ref[...])


def _q_phase(x, q, qb):
    blk = 2000
    return pl.pallas_call(
        _q_body,
        grid=(N // blk,),
        in_specs=[
            pl.BlockSpec((blk, D), lambda i: (i, 0)),
            pl.BlockSpec((D, H), lambda i: (0, 0)),
            pl.BlockSpec((1, H), lambda i: (0, 0)),
        ],
        out_specs=pl.BlockSpec((blk, H), lambda i: (i, 0)),
        out_shape=jax.ShapeDtypeStruct((N, H), jnp.float32),
    )(x, q, qb.reshape(1, H))


def _bcast_lane(vec, lane):
    idx = jnp.full((16, 1), lane, jnp.int32)
    dn = lax.GatherDimensionNumbers(
        offset_dims=(), collapsed_slice_dims=(0,), start_index_map=(0,))
    return lax.gather(vec, idx, dn, (1,),
                      mode=lax.GatherScatterMode.PROMISE_IN_BOUNDS)


def _sc_body(wf_hbm, nbrf_hbm, hid_hbm, out_hbm,
             sidx0, sidx1, raw0, raw1, flat0, flat1, wv0, wv1,
             rows0, rows1, ov0, ov1, hid_sh,
             si0, si1, sw0, sw1, sr0, sr1, so0, so1, sem_h):
    sidx = (sidx0, sidx1)
    raw_v = (raw0, raw1)
    flat_v = (flat0, flat1)
    w_v = (wv0, wv1)
    rows_v = (rows0, rows1)
    out_v = (ov0, ov1)
    sem_i = (si0, si1)
    sem_w = (sw0, sw1)
    sem_r = (sr0, sr1)
    sem_o = (so0, so1)

    c = lax.axis_index("c")
    s = lax.axis_index("s")
    wid = s * NC + c
    base = wid * C

    @pl.when(s == 0)
    def _():
        pltpu.async_copy(hid_hbm, hid_sh, sem_h).wait()

    plsc.subcore_barrier()

    SK = SUP * K

    def idx_dma(m, p):
        pltpu.async_copy(
            nbrf_hbm.at[pl.ds((base + m * SUP) * K, SK)], sidx[p], sem_i[p])

    def idx_wait(p):
        pltpu.make_async_copy(
            nbrf_hbm.at[pl.ds(0, SK)], sidx[p], sem_i[p]).wait()

    def flat_compute(m, p):
        nb = base + m * SUP
        for r in range(SUP):
            roff = (nb + r) * N
            for j in range(K // 16):
                q = r * K + j * 16
                v = sidx[p][pl.ds(q, 16)]
                raw_v[p][pl.ds(q, 16)] = v
                flat_v[p][pl.ds(q, 16)] = jnp.minimum(v + roff, N * N - 1)

    def fire_w(p):
        for g in range(SK // 128):
            sl = pl.ds(g * 128, 128)
            pltpu.async_copy(wf_hbm.at[flat_v[p].at[sl]],
                             w_v[p].at[sl], sem_w[p])

    def wait_w(p):
        for g in range(SK // 128):
            sl = pl.ds(g * 128, 128)
            pltpu.make_async_copy(wf_hbm.at[flat_v[p].at[sl]],
                                  w_v[p].at[sl], sem_w[p]).wait()

    def fire_rows(p, j, gp):
        sl = pl.ds(j * SUB * K, SUB * K)
        pltpu.async_copy(hid_sh.at[raw_v[p].at[sl]], rows_v[gp], sem_r[gp])

    def wait_rows(gp):
        pltpu.make_async_copy(
            hid_sh.at[raw_v[0].at[pl.ds(0, SUB * K)]],
            rows_v[gp], sem_r[gp]).wait()

    def drain_out(p):
        pltpu.make_async_copy(
            out_v[p], out_hbm.at[pl.ds(base, SUP)], sem_o[p]).wait()

    def compute_sub(p, j, gp):
        wait_rows(gp)

        def node(r, carry):
            nq = j * SUB + r
            q = nq * K
            rk = r * K
            w0 = w_v[p][pl.ds(q, 16)]
            w1 = w_v[p][pl.ds(q + 16, 16)]
            acc = [jnp.zeros((16,), jnp.float32) for _ in range(H // 16)]
            wsum = jnp.zeros((16,), jnp.float32)
            for k in range(K):
                wk = _bcast_lane(w0 if k < 16 else w1, k % 16)
                wsum = wsum + wk
                for jj in range(H // 16):
                    acc[jj] = acc[jj] + wk * rows_v[gp][rk + k,
                                                        pl.ds(jj * 16, 16)]
            denom = wsum + 1e-6
            for jj in range(H // 16):
                out_v[p][nq, pl.ds(jj * 16, 16)] = acc[jj] / denom
            return carry

        lax.fori_loop(0, SUB, node, 0)

    def do_super(m, p):
        nb = base + m * SUP

        @pl.when(m >= 2)
        def _():
            drain_out(p)

        fire_rows(p, 0, 0)
        fire_rows(p, 1, 1)
        wait_w(p)
        compute_sub(p, 0, 0)
        fire_rows(p, 2, 0)
        idx_wait(1 - p)
        flat_compute(m + 1, 1 - p)
        fire_w(1 - p)
        compute_sub(p, 1, 1)
        fire_rows(p, 3, 1)
        compute_sub(p, 2, 0)
        compute_sub(p, 3, 1)
        pltpu.async_copy(out_v[p], out_hbm.at[pl.ds(nb, SUP)], sem_o[p])
        idx_dma(m + 2, p)

    idx_dma(0, 0)
    idx_wait(0)
    flat_compute(0, 0)
    fire_w(0)
    idx_dma(1, 1)

    def body(i, carry):
        do_super(2 * i, 0)
        do_super(2 * i + 1, 1)
        return carry

    lax.fori_loop(0, NSUP // 2, body, 0)
    wait_w(0)
    idx_wait(1)
    drain_out(0)
    drain_out(1)


def _sc_phase(weights_flat, nbr_flat, hid):
    mesh = plsc.VectorSubcoreMesh(core_axis_name="c", subcore_axis_name="s")
    scratch = (
        [pltpu.VMEM((SUP * K,), jnp.int32) for _ in range(2)]
        + [pltpu.VMEM((SUP * K,), jnp.int32) for _ in range(4)]
        + [pltpu.VMEM((SUP * K,), jnp.float32) for _ in range(2)]
        + [pltpu.VMEM((SUB * K, H), jnp.float32) for _ in range(2)]
        + [pltpu.VMEM((SUP, H), jnp.float32) for _ in range(2)]
        + [pltpu.VMEM_SHARED((N, H), jnp.float32)]
        + [pltpu.SemaphoreType.DMA for _ in range(9)]
    )
    f = pl.kernel(
        _sc_body,
        out_type=jax.ShapeDtypeStruct((NP, H), jnp.float32),
        mesh=mesh,
        scratch_types=scratch,
    )
    return f(weights_flat, nbr_flat, hid)


def _w_body(x_ref, a_ref, w1_ref, w2_ref, b_ref, o_ref):
    h = jnp.dot(x_ref[...], w1_ref[...], preferred_element_type=jnp.float32)
    h = h + jnp.dot(a_ref[...], w2_ref[...],
                    preferred_element_type=jnp.float32)
    h = _leaky(h + b_ref[...])
    nrm = jnp.sqrt(jnp.sum(h * h, axis=1, keepdims=True)) + 1e-6
    o_ref[...] = h / nrm


def _w_phase(x, agg, w1, w2, wb):
    blk = 2000
    return pl.pallas_call(
        _w_body,
        grid=(N // blk,),
        in_specs=[
            pl.BlockSpec((blk, D), lambda i: (i, 0)),
            pl.BlockSpec((blk, H), lambda i: (i, 0)),
            pl.BlockSpec((D, H), lambda i: (0, 0)),
            pl.BlockSpec((H, H), lambda i: (0, 0)),
            pl.BlockSpec((1, H), lambda i: (0, 0)),
        ],
        out_specs=pl.BlockSpec((blk, H), lambda i: (i, 0)),
        out_shape=jax.ShapeDtypeStruct((N, H), jnp.float32),
    )(x, agg, w1, w2, wb.reshape(1, H))


def kernel(embs, weights, Q_kernel, Q_bias, W_kernel, W_bias, neighbor_set):
    x = embs[0]
    hid = _q_phase(x, Q_kernel, Q_bias)
    wf = weights.reshape(N * N)
    nbr = jnp.pad(neighbor_set.astype(jnp.int32),
                  ((0, NP + 2 * SUP - N), (0, 0)))
    nbrf = nbr.reshape((NP + 2 * SUP) * K)
    agg = _sc_phase(wf, nbrf, hid)[:N]
    out = _w_phase(x, agg, W_kernel[:D], W_kernel[D:], W_bias)
    return out[None]
